# baseline (speedup 1.0000x reference)
import functools
import jax
import jax.numpy as jnp
from jax import lax
from jax.experimental import pallas as pl
from jax.experimental.pallas import tpu as pltpu
from jax.experimental.pallas import tpu_sc as plsc

N = 10000
E = 320000
D = 128
NCLS = 40
EPS = 1.0

NPAD = 10240
EPAD = 327680
NTILE = 32
EPT = EPAD // NTILE
CH = 128
NCHUNK = EPT // CH
RPT = NPAD // 16
NBLK = NPAD // 128

_SC_MESH = plsc.VectorSubcoreMesh(
    core_axis_name="c", subcore_axis_name="s", num_cores=2, num_subcores=16
)
_SC_PARAMS = pltpu.CompilerParams(needs_layout_passes=False)


@functools.partial(
    pl.kernel,
    out_type=jax.ShapeDtypeStruct((NTILE, NPAD), jnp.float32),
    mesh=_SC_MESH,
    scratch_types=[
        pltpu.VMEM((NPAD,), jnp.float32),
        pltpu.VMEM((EPT,), jnp.int32),
    ],
    compiler_params=_SC_PARAMS,
)
def _sc_degree(dst_hbm, out_hbm, hist, dstv):
    cid = lax.axis_index("c")
    sid = lax.axis_index("s")
    wid = cid * 16 + sid

    zeros16 = jnp.zeros((16,), jnp.float32)

    def zero_body(i, _):
        hist[pl.ds(i * 16, 16)] = zeros16
        return 0

    lax.fori_loop(0, NPAD // 16, zero_body, 0)

    pltpu.sync_copy(dst_hbm.at[wid], dstv)

    ones16 = jnp.ones((16,), jnp.float32)

    def acc_body(i, _):
        idx = dstv[pl.ds(i * 16, 16)]
        plsc.addupdate_scatter(hist, [idx], ones16)
        return 0

    lax.fori_loop(0, EPT // 16, acc_body, 0)

    pltpu.sync_copy(hist, out_hbm.at[wid])


RBLK = 8
NSTEP = NBLK // RBLK


def _h2_body(x_ref, w_ref, b_ref, deg_ref, h2_ref):
    d = jnp.sum(deg_ref[...], axis=2, keepdims=True) + 1.0
    dinv = lax.rsqrt(d)
    xb = x_ref[...].reshape(RBLK * 128, D)
    h = jnp.dot(xb, w_ref[...], preferred_element_type=jnp.float32)
    h = h + b_ref[...]
    h2_ref[...] = h.reshape(RBLK, 128, D) * dinv


_tc_h2 = pl.pallas_call(
    _h2_body,
    grid=(NSTEP,),
    in_specs=[
        pl.BlockSpec((RBLK, 128, D), lambda i: (i, 0, 0)),
        pl.BlockSpec((D, D), lambda i: (0, 0)),
        pl.BlockSpec((1, D), lambda i: (0, 0)),
        pl.BlockSpec((RBLK, 128, NTILE), lambda i: (i, 0, 0)),
    ],
    out_specs=pl.BlockSpec((RBLK, 128, D), lambda i: (i, 0, 0)),
    out_shape=jax.ShapeDtypeStruct((NBLK, 128, D), jnp.float32),
)


@functools.partial(
    pl.kernel,
    out_type=jax.ShapeDtypeStruct((2, NPAD, D), jnp.float32),
    mesh=_SC_MESH,
    scratch_types=[
        pltpu.VMEM((NCHUNK, CH), jnp.int32),
        pltpu.VMEM((1, CH), jnp.int32),
        pltpu.VMEM((1, CH), jnp.int32),
        pltpu.VMEM((CH, D), jnp.float32),
        pltpu.VMEM((CH, D), jnp.float32),
        pltpu.VMEM_SHARED((NPAD, D), jnp.float32),
        pltpu.SemaphoreType.DMA,
        pltpu.SemaphoreType.DMA,
        pltpu.SemaphoreType.DMA,
        pltpu.SemaphoreType.DMA,
    ],
    compiler_params=_SC_PARAMS,
)
def _sc_scatter(src_hbm, dst_hbm, h2_hbm, zero_hbm, out_hbm,
                srcv, db0, db1, buf0, buf1, acc, sem0, sem1, dsem0, dsem1):
    cid = lax.axis_index("c")
    sid = lax.axis_index("s")
    wid = cid * 16 + sid

    pltpu.sync_copy(zero_hbm.at[pl.ds(sid * RPT, RPT)],
                    acc.at[pl.ds(sid * RPT, RPT)])
    pltpu.sync_copy(src_hbm.at[wid], srcv)
    plsc.subcore_barrier()

    def g_start(c, buf, sem):
        pltpu.make_async_copy(h2_hbm.at[srcv.at[c]], buf, sem).start()

    def g_wait(c, buf, sem):
        pltpu.make_async_copy(h2_hbm.at[srcv.at[c]], buf, sem).wait()

    def d_start(c, db, sem):
        pltpu.make_async_copy(dst_hbm.at[wid, c], db.at[0], sem).start()

    def d_wait(c, db, sem):
        pltpu.make_async_copy(dst_hbm.at[wid, c], db.at[0], sem).wait()

    g_start(0, buf0, sem0)
    d_start(0, db0, dsem0)

    def body(i, _):
        c0 = 2 * i
        c1 = c0 + 1
        g_start(c1, buf1, sem1)
        d_start(c1, db1, dsem1)
        g_wait(c0, buf0, sem0)
        d_wait(c0, db0, dsem0)
        pltpu.sync_copy(buf0, acc.at[db0.at[0]], add=True)

        @pl.when(i < NCHUNK // 2 - 1)
        def _():
            g_start(c0 + 2, buf0, sem0)
            d_start(c0 + 2, db0, dsem0)

        g_wait(c1, buf1, sem1)
        d_wait(c1, db1, dsem1)
        pltpu.sync_copy(buf1, acc.at[db1.at[0]], add=True)
        return 0

    lax.fori_loop(0, NCHUNK // 2, body, 0)
    plsc.subcore_barrier()

    pltpu.sync_copy(acc.at[pl.ds(sid * RPT, RPT)],
                    out_hbm.at[cid, pl.ds(sid * RPT, RPT), :])


def _final_body(p0_ref, p1_ref, h2_ref, deg_ref, wro_ref, bro_ref, out_ref):
    d = jnp.sum(deg_ref[...], axis=2, keepdims=True) + 1.0
    dinv = lax.rsqrt(d)
    s = p0_ref[0] + p1_ref[0] + EPS * h2_ref[...]
    o = jnp.maximum(s * dinv, 0.0).reshape(RBLK * 128, D)
    logits = jnp.dot(o, wro_ref[...], preferred_element_type=jnp.float32)
    logits = logits + bro_ref[...]
    m = jnp.max(logits, axis=1, keepdims=True)
    e = jnp.exp(logits - m)
    lse = jnp.log(jnp.sum(e, axis=1, keepdims=True))
    out_ref[...] = logits - m - lse


_tc_final = pl.pallas_call(
    _final_body,
    grid=(NSTEP,),
    in_specs=[
        pl.BlockSpec((1, RBLK, 128, D), lambda i: (0, i, 0, 0)),
        pl.BlockSpec((1, RBLK, 128, D), lambda i: (1, i, 0, 0)),
        pl.BlockSpec((RBLK, 128, D), lambda i: (i, 0, 0)),
        pl.BlockSpec((RBLK, 128, NTILE), lambda i: (i, 0, 0)),
        pl.BlockSpec((D, D), lambda i: (0, 0)),
        pl.BlockSpec((1, D), lambda i: (0, 0)),
    ],
    out_specs=pl.BlockSpec((RBLK * 128, D), lambda i: (i, 0)),
    out_shape=jax.ShapeDtypeStruct((NPAD, D), jnp.float32),
)


def kernel(x, edge_index, W1, b1, W_ro, b_ro):
    src = edge_index[0]
    dst = edge_index[1]
    pad_e = EPAD - E
    fill = jnp.arange(pad_e, dtype=jnp.int32)
    src_p = jnp.concatenate([src, fill % N])
    dst_p = jnp.concatenate([dst, N + fill % (NPAD - N)])
    src3 = src_p.reshape(NTILE, NCHUNK, CH)
    dst3 = dst_p.reshape(NTILE, NCHUNK, CH)
    dst_flat = dst_p.reshape(NTILE, EPT)

    x_pad = jnp.pad(x, ((0, NPAD - N), (0, 0)))
    b1r = b1.reshape(1, D)
    wro_pad = jnp.pad(W_ro, ((0, 0), (0, D - NCLS)))
    bro_pad = jnp.concatenate(
        [b_ro, jnp.full((D - NCLS,), -1e30, jnp.float32)]
    ).reshape(1, D)
    zero_acc = jnp.zeros((NPAD, D), jnp.float32)

    deg_parts = _sc_degree(dst_flat)
    deg_t = deg_parts.reshape(NTILE, NBLK, 128).transpose(1, 2, 0)

    x_blk = x_pad.reshape(NBLK, 128, D)
    h2 = _tc_h2(x_blk, W1, b1r, deg_t)
    parts = _sc_scatter(src3, dst3, h2.reshape(NPAD, D), zero_acc)
    parts_blk = parts.reshape(2, NBLK, 128, D)
    res = _tc_final(parts_blk, parts_blk, h2, deg_t, wro_pad, bro_pad)
    return res[:N, :NCLS]


# speedup vs baseline: 1.0062x; 1.0062x over previous
import functools
import jax
import jax.numpy as jnp
from jax import lax
from jax.experimental import pallas as pl
from jax.experimental.pallas import tpu as pltpu
from jax.experimental.pallas import tpu_sc as plsc

N = 10000
E = 320000
D = 128
NCLS = 40
EPS = 1.0

NPAD = 10240
NTILE = 32
CH = 128
CTOT = E // CH
MAXC = 80
RPT = NPAD // 16
NBLK = NPAD // 128

_SC_MESH = plsc.VectorSubcoreMesh(
    core_axis_name="c", subcore_axis_name="s", num_cores=2, num_subcores=16
)
_SC_PARAMS = pltpu.CompilerParams(needs_layout_passes=False)


@functools.partial(
    pl.kernel,
    out_type=jax.ShapeDtypeStruct((NTILE, NPAD), jnp.float32),
    mesh=_SC_MESH,
    scratch_types=[
        pltpu.VMEM((NPAD,), jnp.float32),
        pltpu.VMEM((MAXC, CH), jnp.int32),
    ],
    compiler_params=_SC_PARAMS,
)
def _sc_degree(dst_hbm, out_hbm, hist, dstv):
    cid = lax.axis_index("c")
    sid = lax.axis_index("s")
    wid = cid * 16 + sid
    start = 80 * wid - 8 * (wid // 4)
    is_big = (wid % 4) < 3
    count = jnp.where(is_big, 80, 72) + jnp.where(wid == NTILE - 1, 4, 0)

    zeros16 = jnp.zeros((16,), jnp.float32)

    def zero_body(i, _):
        hist[pl.ds(i * 16, 16)] = zeros16
        return 0

    lax.fori_loop(0, NPAD // 16, zero_body, 0)

    pltpu.sync_copy(dst_hbm.at[pl.ds(start, 72)], dstv.at[pl.ds(0, 72)])

    @pl.when(is_big)
    def _():
        pltpu.sync_copy(dst_hbm.at[pl.ds(start + 72, 8)],
                        dstv.at[pl.ds(72, 8)])

    @pl.when(wid == NTILE - 1)
    def _():
        pltpu.sync_copy(dst_hbm.at[pl.ds(start + 72, 4)],
                        dstv.at[pl.ds(72, 4)])

    ones16 = jnp.ones((16,), jnp.float32)

    def acc_body(i, _):
        idx = dstv[i // 8, pl.ds((i % 8) * 16, 16)]
        plsc.addupdate_scatter(hist, [idx], ones16)
        return 0

    lax.fori_loop(0, count * 8, acc_body, 0)

    pltpu.sync_copy(hist, out_hbm.at[wid])


RBLK = 8
NSTEP = NBLK // RBLK


def _h2_body(x_ref, w_ref, b_ref, deg_ref, h2_ref):
    d = jnp.sum(deg_ref[...], axis=2, keepdims=True) + 1.0
    dinv = lax.rsqrt(d)
    xb = x_ref[...].reshape(RBLK * 128, D)
    h = jnp.dot(xb, w_ref[...], preferred_element_type=jnp.float32)
    h = h + b_ref[...]
    h2_ref[...] = h.reshape(RBLK, 128, D) * dinv


_tc_h2 = pl.pallas_call(
    _h2_body,
    grid=(NSTEP,),
    in_specs=[
        pl.BlockSpec((RBLK, 128, D), lambda i: (i, 0, 0)),
        pl.BlockSpec((D, D), lambda i: (0, 0)),
        pl.BlockSpec((1, D), lambda i: (0, 0)),
        pl.BlockSpec((RBLK, 128, NTILE), lambda i: (i, 0, 0)),
    ],
    out_specs=pl.BlockSpec((RBLK, 128, D), lambda i: (i, 0, 0)),
    out_shape=jax.ShapeDtypeStruct((NBLK, 128, D), jnp.float32),
)


@functools.partial(
    pl.kernel,
    out_type=jax.ShapeDtypeStruct((2, NPAD, D), jnp.float32),
    mesh=_SC_MESH,
    scratch_types=[
        pltpu.VMEM((MAXC, CH), jnp.int32),
        pltpu.VMEM((1, CH), jnp.int32),
        pltpu.VMEM((1, CH), jnp.int32),
        pltpu.VMEM((CH, D), jnp.float32),
        pltpu.VMEM((CH, D), jnp.float32),
        pltpu.VMEM_SHARED((NPAD, D), jnp.float32),
        pltpu.SemaphoreType.DMA,
        pltpu.SemaphoreType.DMA,
        pltpu.SemaphoreType.DMA,
        pltpu.SemaphoreType.DMA,
    ],
    compiler_params=_SC_PARAMS,
)
def _sc_scatter(src_hbm, dst_hbm, h2_hbm, zero_hbm, out_hbm,
                srcv, db0, db1, buf0, buf1, acc, sem0, sem1, dsem0, dsem1):
    cid = lax.axis_index("c")
    sid = lax.axis_index("s")
    wid = cid * 16 + sid
    start = 80 * wid - 8 * (wid // 4)
    is_big = (wid % 4) < 3
    npair = jnp.where(is_big, 40, 36) + jnp.where(wid == NTILE - 1, 2, 0)

    pltpu.sync_copy(zero_hbm.at[pl.ds(sid * RPT, RPT)],
                    acc.at[pl.ds(sid * RPT, RPT)])
    pltpu.sync_copy(src_hbm.at[pl.ds(start, 72)], srcv.at[pl.ds(0, 72)])

    @pl.when(is_big)
    def _():
        pltpu.sync_copy(src_hbm.at[pl.ds(start + 72, 8)],
                        srcv.at[pl.ds(72, 8)])

    @pl.when(wid == NTILE - 1)
    def _():
        pltpu.sync_copy(src_hbm.at[pl.ds(start + 72, 4)],
                        srcv.at[pl.ds(72, 4)])

    plsc.subcore_barrier()

    def g_start(c, buf, sem):
        pltpu.make_async_copy(h2_hbm.at[srcv.at[c]], buf, sem).start()

    def g_wait(c, buf, sem):
        pltpu.make_async_copy(h2_hbm.at[srcv.at[c]], buf, sem).wait()

    def d_start(c, db, sem):
        pltpu.make_async_copy(dst_hbm.at[start + c], db.at[0], sem).start()

    def d_wait(c, db, sem):
        pltpu.make_async_copy(dst_hbm.at[start + c], db.at[0], sem).wait()

    g_start(0, buf0, sem0)
    d_start(0, db0, dsem0)

    def body(i, _):
        c0 = 2 * i
        c1 = c0 + 1
        g_start(c1, buf1, sem1)
        d_start(c1, db1, dsem1)
        g_wait(c0, buf0, sem0)
        d_wait(c0, db0, dsem0)
        pltpu.sync_copy(buf0, acc.at[db0.at[0]], add=True)

        @pl.when(i < npair - 1)
        def _():
            g_start(c0 + 2, buf0, sem0)
            d_start(c0 + 2, db0, dsem0)

        g_wait(c1, buf1, sem1)
        d_wait(c1, db1, dsem1)
        pltpu.sync_copy(buf1, acc.at[db1.at[0]], add=True)
        return 0

    lax.fori_loop(0, npair, body, 0)
    plsc.subcore_barrier()

    pltpu.sync_copy(acc.at[pl.ds(sid * RPT, RPT)],
                    out_hbm.at[cid, pl.ds(sid * RPT, RPT), :])


def _final_body(p0_ref, p1_ref, h2_ref, deg_ref, wro_ref, bro_ref, out_ref):
    d = jnp.sum(deg_ref[...], axis=2, keepdims=True) + 1.0
    dinv = lax.rsqrt(d)
    s = p0_ref[0] + p1_ref[0] + EPS * h2_ref[...]
    o = jnp.maximum(s * dinv, 0.0).reshape(RBLK * 128, D)
    logits = jnp.dot(o, wro_ref[...], preferred_element_type=jnp.float32)
    logits = logits + bro_ref[...]
    m = jnp.max(logits, axis=1, keepdims=True)
    e = jnp.exp(logits - m)
    lse = jnp.log(jnp.sum(e, axis=1, keepdims=True))
    out_ref[...] = logits - m - lse


_tc_final = pl.pallas_call(
    _final_body,
    grid=(NSTEP,),
    in_specs=[
        pl.BlockSpec((1, RBLK, 128, D), lambda i: (0, i, 0, 0)),
        pl.BlockSpec((1, RBLK, 128, D), lambda i: (1, i, 0, 0)),
        pl.BlockSpec((RBLK, 128, D), lambda i: (i, 0, 0)),
        pl.BlockSpec((RBLK, 128, NTILE), lambda i: (i, 0, 0)),
        pl.BlockSpec((D, D), lambda i: (0, 0)),
        pl.BlockSpec((1, D), lambda i: (0, 0)),
    ],
    out_specs=pl.BlockSpec((RBLK * 128, D), lambda i: (i, 0)),
    out_shape=jax.ShapeDtypeStruct((NPAD, D), jnp.float32),
)


def kernel(x, edge_index, W1, b1, W_ro, b_ro):
    src2 = edge_index[0].reshape(CTOT, CH)
    dst2 = edge_index[1].reshape(CTOT, CH)

    x_pad = jnp.pad(x, ((0, NPAD - N), (0, 0)))
    b1r = b1.reshape(1, D)
    wro_pad = jnp.pad(W_ro, ((0, 0), (0, D - NCLS)))
    bro_pad = jnp.concatenate(
        [b_ro, jnp.full((D - NCLS,), -1e30, jnp.float32)]
    ).reshape(1, D)
    zero_acc = jnp.zeros((NPAD, D), jnp.float32)

    deg_parts = _sc_degree(dst2)
    deg_t = deg_parts.reshape(NTILE, NBLK, 128).transpose(1, 2, 0)

    x_blk = x_pad.reshape(NBLK, 128, D)
    h2 = _tc_h2(x_blk, W1, b1r, deg_t)
    parts = _sc_scatter(src2, dst2, h2.reshape(NPAD, D), zero_acc)
    parts_blk = parts.reshape(2, NBLK, 128, D)
    res = _tc_final(parts_blk, parts_blk, h2, deg_t, wro_pad, bro_pad)
    return res[:N, :NCLS]
